# baseline (speedup 1.0000x reference)
import functools

import jax
import jax.numpy as jnp
from jax import lax
from jax.experimental import pallas as pl
from jax.experimental.pallas import tpu as pltpu
from jax.experimental.pallas import tpu_sc as plsc

B, S, HIDDEN = 1, 2048, 768
NH, HD = 12, 64
E, EXP = 8, 768
EPS = 1e-05
SCALE = 1.0 / 8.0

TROW = 256
QBLK = 256
KBLK = 256
TBLK = 256
NB = S // TBLK
STEPS = NB + E - 1

_F32 = jnp.float32
_BF16 = jnp.bfloat16
_I32 = jnp.int32

_NT = (((1,), (1,)), ((), ()))


def _qkv_body(x_ref, ln1_ref, wq_ref, wk_ref, wv_ref, q_ref, k_ref, v_ref,
              wcast):
    @pl.when(pl.program_id(0) == 0)
    def _cast_once():
        for m, w_ref in enumerate((wq_ref, wk_ref, wv_ref)):
            wcast[m] = w_ref[...].astype(_BF16)

    x = x_ref[...]
    var = jnp.mean(jnp.square(x), axis=1, keepdims=True)
    h = (x * lax.rsqrt(var + EPS) * ln1_ref[...]).astype(_BF16)
    for m, o_ref in enumerate((q_ref, k_ref, v_ref)):
        o_ref[...] = lax.dot_general(
            h, wcast[m], _NT, preferred_element_type=_F32).astype(_BF16)


def _attn_body(pair, width):
    def body(q_ref, k_ref, v_ref, o_ref):
        base = (pair * 2 + pl.program_id(0)) * QBLK
        rows = lax.broadcasted_iota(_I32, (QBLK, width), 0) + base
        cols = lax.broadcasted_iota(_I32, (QBLK, width), 1)
        bias = jnp.where(cols > rows, -jnp.inf, 0.0).astype(_F32)
        qs = q_ref[...] * jnp.asarray(SCALE, _BF16)
        for h in range(NH):
            sl = slice(h * HD, (h + 1) * HD)
            sc = lax.dot_general(qs[:, sl], k_ref[:, sl], _NT,
                                 preferred_element_type=_F32)
            sc = sc + bias
            m = jnp.max(sc, axis=1, keepdims=True)
            e = jnp.exp(sc - m)
            r = 1.0 / jnp.sum(e, axis=1, keepdims=True)
            o = lax.dot_general((e * r).astype(_BF16), v_ref[:, sl],
                                (((1,), (0,)), ((), ())),
                                preferred_element_type=_F32)
            o_ref[:, sl] = o.astype(_BF16)

    return body


def _post_body(o_ref, x_ref, wo_ref, ln2_ref, rw_ref, x1_ref, idx_ref,
               wcast):
    @pl.when(pl.program_id(0) == 0)
    def _cast_once():
        wcast[0] = wo_ref[...].astype(_BF16)

    x1 = x_ref[...] + lax.dot_general(
        o_ref[...], wcast[0], _NT,
        preferred_element_type=_F32)
    x1_ref[...] = x1
    var = jnp.mean(jnp.square(x1), axis=1, keepdims=True)
    h2 = (x1 * lax.rsqrt(var + EPS) * ln2_ref[...]).astype(_BF16)
    logits = lax.dot_general(h2, rw_ref[...].astype(_BF16), _NT,
                             preferred_element_type=_F32)
    mx = jnp.max(logits, axis=1, keepdims=True)
    eidx = lax.broadcasted_iota(_I32, logits.shape, 1)
    idx_ref[...] = jnp.min(jnp.where(logits == mx, eidx, E), axis=1)


def _moe_body(sb_ref, se_ref, lo_ref, hi_ref, nxe_ref, buf_ref,
              xs_ref, ln2_ref, wg_ref, wu_ref, wd_ref, out_ref,
              wbuf, wcast, wsem):
    s = pl.program_id(0)
    lo = lo_ref[s]
    hi = hi_ref[s]
    b = buf_ref[s]
    e = se_ref[s]
    nxe = nxe_ref[s]
    first = jnp.logical_or(s == 0, sb_ref[s] != sb_ref[jnp.maximum(s - 1, 0)])
    is_chg = jnp.logical_or(
        s == 0, se_ref[jnp.maximum(s - 1, 0)] != e)
    hbm = (wg_ref, wu_ref, wd_ref)

    @pl.when(s == 0)
    def _start_first():
        for m in range(3):
            pltpu.make_async_copy(hbm[m].at[e], wbuf.at[0, m],
                                  wsem.at[0, m]).start()

    @pl.when(is_chg)
    def _wait_and_prefetch():
        for m in range(3):
            pltpu.make_async_copy(hbm[m].at[e], wbuf.at[b, m],
                                  wsem.at[b, m]).wait()
        for m in range(3):
            wcast[b, m] = wbuf[b, m].astype(_BF16)

    @pl.when(jnp.logical_and(is_chg, nxe != e))
    def _prefetch_next():
        for m in range(3):
            pltpu.make_async_copy(hbm[m].at[nxe], wbuf.at[1 - b, m],
                                  wsem.at[1 - b, m]).start()

    @pl.when(first)
    def _init():
        out_ref[...] = jnp.zeros_like(out_ref)

    @pl.when(hi > lo)
    def _compute():
        x = xs_ref[...]
        r = lax.broadcasted_iota(_I32, (TBLK, 1), 0)
        msk = jnp.logical_and(r >= lo, r < hi)
        xm = jnp.where(msk, x, 0.0)
        var = jnp.mean(jnp.square(xm), axis=1, keepdims=True)
        h2 = (xm * lax.rsqrt(var + EPS) * ln2_ref[...]).astype(_BF16)
        g = lax.dot_general(h2, wcast[b, 0], _NT,
                            preferred_element_type=_F32)
        g = jnp.square(jnp.maximum(g, 0.0))
        u = lax.dot_general(h2, wcast[b, 1], _NT,
                            preferred_element_type=_F32)
        p = (g * u).astype(_BF16)
        y = lax.dot_general(p, wcast[b, 2], _NT,
                            preferred_element_type=_F32)
        out_ref[...] += jnp.where(msk, y + x, 0.0)


def _row_permute_kernel(scatter):
    info = plsc.get_sparse_core_info()
    nc, ns = info.num_cores, info.num_subcores
    nw = nc * ns
    rpw = S // nw
    mesh = plsc.VectorSubcoreMesh(core_axis_name="c", subcore_axis_name="s")

    @functools.partial(
        pl.kernel,
        out_type=jax.ShapeDtypeStruct((S, HIDDEN), _F32),
        mesh=mesh,
        scratch_types=[
            pltpu.VMEM((rpw,), _I32),
            pltpu.VMEM((rpw, HIDDEN), _F32),
            pltpu.SemaphoreType.DMA,
        ],
    )
    def permute(table_hbm, idx_hbm, out_hbm, idx_v, rows_v, sem):
        wid = lax.axis_index("s") * nc + lax.axis_index("c")
        base = wid * rpw
        pltpu.sync_copy(idx_hbm.at[pl.ds(base, rpw)], idx_v)
        if scatter:
            pltpu.sync_copy(table_hbm.at[pl.ds(base, rpw)], rows_v)
            pltpu.async_copy(rows_v, out_hbm.at[idx_v], sem).wait()
        else:
            pltpu.async_copy(table_hbm.at[idx_v], rows_v, sem).wait()
            pltpu.sync_copy(rows_v, out_hbm.at[pl.ds(base, rpw)])

    return permute


def _routing_metadata(idx):
    oh = (idx[:, None] == jnp.arange(E, dtype=_I32)[None, :]).astype(_I32)
    csum = jnp.cumsum(oh, axis=0)
    within = jnp.sum((csum - oh) * oh, axis=1)
    counts = csum[-1]
    offsets = jnp.concatenate(
        [jnp.zeros((1,), _I32), jnp.cumsum(counts).astype(_I32)])
    dest = jnp.take(offsets, idx) + within

    bb = jnp.arange(NB, dtype=_I32)[:, None]
    ee = jnp.arange(E, dtype=_I32)[None, :]
    st = offsets[:-1][None, :]
    en = offsets[1:][None, :]
    active = jnp.logical_and(st < (bb + 1) * TBLK, en > bb * TBLK)
    big = NB * E + 1
    key = jnp.where(active, bb * E + ee, big).reshape(-1)
    key = jnp.sort(key)[:STEPS]
    valid = key < big
    key = jnp.where(valid, key, jnp.max(jnp.where(active, bb * E + ee, -1)))
    sb = (key // E).astype(_I32)
    se = (key % E).astype(_I32)
    lo = jnp.clip(jnp.take(offsets, se) - sb * TBLK, 0, TBLK)
    hi = jnp.clip(jnp.take(offsets, se + 1) - sb * TBLK, 0, TBLK)
    lo = jnp.where(valid, lo, 0).astype(_I32)
    hi = jnp.where(valid, hi, 0).astype(_I32)

    chg = jnp.concatenate([jnp.ones((1,), _I32),
                           (se[1:] != se[:-1]).astype(_I32)])
    buf = ((jnp.cumsum(chg) - 1) % 2).astype(_I32)
    pos = jnp.where(chg == 1, jnp.arange(STEPS, dtype=_I32), STEPS)
    nxt = lax.cummin(jnp.concatenate([pos[1:], jnp.full((1,), STEPS, _I32)]),
                     axis=0, reverse=True)
    nxe = jnp.where(nxt < STEPS, se[jnp.clip(nxt, 0, STEPS - 1)], se)
    return dest, sb, se, lo, hi, nxe.astype(_I32), buf


def kernel(x, Wq, Wk, Wv, Wo, ln1, ln2, router_w, Wg, Wu, Wd):
    x2 = x.reshape(S, HIDDEN)
    ln1r = ln1.reshape(1, HIDDEN)
    ln2r = ln2.reshape(1, HIDDEN)

    row_spec = pl.BlockSpec((TROW, HIDDEN), lambda i: (i, 0))
    vec_spec = pl.BlockSpec((1, HIDDEN), lambda i: (0, 0))
    full_w = pl.BlockSpec((HIDDEN, HIDDEN), lambda i: (0, 0))

    q, k, v = pl.pallas_call(
        _qkv_body,
        grid=(S // TROW,),
        in_specs=[row_spec, vec_spec, full_w, full_w, full_w],
        out_specs=[pl.BlockSpec((TROW, HIDDEN), lambda i: (i, 0))] * 3,
        out_shape=[jax.ShapeDtypeStruct((S, HIDDEN), _BF16)] * 3,
        scratch_shapes=[pltpu.VMEM((3, HIDDEN, HIDDEN), _BF16)],
    )(x2, ln1r, Wq, Wk, Wv)

    o_parts = []
    for pair in range(S // (2 * QBLK)):
        width = (pair + 1) * 2 * QBLK
        o_parts.append(pl.pallas_call(
            _attn_body(pair, width),
            grid=(2,),
            in_specs=[
                pl.BlockSpec((QBLK, HIDDEN),
                             lambda i, p=pair: (2 * p + i, 0)),
                pl.BlockSpec((width, HIDDEN), lambda i: (0, 0)),
                pl.BlockSpec((width, HIDDEN), lambda i: (0, 0)),
            ],
            out_specs=pl.BlockSpec((QBLK, HIDDEN), lambda i: (i, 0)),
            out_shape=jax.ShapeDtypeStruct((2 * QBLK, HIDDEN), _BF16),
        )(q, k, v))
    o = jnp.concatenate(o_parts, axis=0)

    x1, idx = pl.pallas_call(
        _post_body,
        grid=(S // TROW,),
        in_specs=[row_spec, row_spec, full_w, vec_spec,
                  pl.BlockSpec((E, HIDDEN), lambda i: (0, 0))],
        out_specs=[row_spec, pl.BlockSpec((TROW,), lambda i: (i,))],
        out_shape=[jax.ShapeDtypeStruct((S, HIDDEN), _F32),
                   jax.ShapeDtypeStruct((S,), _I32)],
        scratch_shapes=[pltpu.VMEM((1, HIDDEN, HIDDEN), _BF16)],
    )(o, x2, Wo, ln2r, router_w)

    dest, sb, se, lo, hi, nxe, buf = _routing_metadata(idx)

    xs = _row_permute_kernel(scatter=True)(x1, dest)

    grid_spec = pltpu.PrefetchScalarGridSpec(
        num_scalar_prefetch=6,
        grid=(STEPS,),
        in_specs=[
            pl.BlockSpec((TBLK, HIDDEN),
                         lambda s, sb, se, lo, hi, nxe, buf: (sb[s], 0)),
            pl.BlockSpec((1, HIDDEN),
                         lambda s, sb, se, lo, hi, nxe, buf: (0, 0)),
            pl.BlockSpec(memory_space=pl.ANY),
            pl.BlockSpec(memory_space=pl.ANY),
            pl.BlockSpec(memory_space=pl.ANY),
        ],
        out_specs=pl.BlockSpec((TBLK, HIDDEN),
                               lambda s, sb, se, lo, hi, nxe, buf: (sb[s], 0)),
        scratch_shapes=[
            pltpu.VMEM((2, 3, HIDDEN, HIDDEN), _F32),
            pltpu.VMEM((2, 3, HIDDEN, HIDDEN), _BF16),
            pltpu.SemaphoreType.DMA((2, 3)),
        ],
    )
    ys = pl.pallas_call(
        _moe_body,
        grid_spec=grid_spec,
        out_shape=jax.ShapeDtypeStruct((S, HIDDEN), _F32),
    )(sb, se, lo, hi, nxe, buf, xs, ln2r, Wg, Wu, Wd)

    out = _row_permute_kernel(scatter=False)(ys, dest)
    return out.reshape(B, S, HIDDEN)


# speedup vs baseline: 1.0149x; 1.0149x over previous
import functools

import jax
import jax.numpy as jnp
from jax import lax
from jax.experimental import pallas as pl
from jax.experimental.pallas import tpu as pltpu
from jax.experimental.pallas import tpu_sc as plsc

B, S, HIDDEN = 1, 2048, 768
NH, HD = 12, 64
E, EXP = 8, 768
EPS = 1e-05
SCALE = 1.0 / 8.0

TROW = 256
QBLK = 256
KBLK = 256
TBLK = 256
NB = S // TBLK
STEPS = NB + E - 1

_F32 = jnp.float32
_BF16 = jnp.bfloat16
_I32 = jnp.int32

_NT = (((1,), (1,)), ((), ()))


def _qkv_body(x_ref, ln1_ref, wq_ref, wk_ref, wv_ref, q_ref, k_ref, v_ref):
    x = x_ref[...]
    var = jnp.mean(jnp.square(x), axis=1, keepdims=True)
    h = (x * lax.rsqrt(var + EPS) * ln1_ref[...]).astype(_BF16)
    for w_ref, o_ref in ((wq_ref, q_ref), (wk_ref, k_ref), (wv_ref, v_ref)):
        o_ref[...] = lax.dot_general(
            h, w_ref[...].astype(_BF16), _NT,
            preferred_element_type=_F32).astype(_BF16)


def _attn_body(pair, width):
    def body(q_ref, k_ref, v_ref, o_ref):
        base = (pair * 2 + pl.program_id(0)) * QBLK
        rows = lax.broadcasted_iota(_I32, (QBLK, width), 0) + base
        cols = lax.broadcasted_iota(_I32, (QBLK, width), 1)
        bias = jnp.where(cols > rows, -jnp.inf, 0.0).astype(_F32)
        qs = q_ref[...] * jnp.asarray(SCALE, _BF16)
        for h in range(NH):
            sl = slice(h * HD, (h + 1) * HD)
            sc = lax.dot_general(qs[:, sl], k_ref[:, sl], _NT,
                                 preferred_element_type=_F32)
            sc = sc + bias
            m = jnp.max(sc, axis=1, keepdims=True)
            e = jnp.exp(sc - m)
            r = 1.0 / jnp.sum(e, axis=1, keepdims=True)
            o = lax.dot_general((e * r).astype(_BF16), v_ref[:, sl],
                                (((1,), (0,)), ((), ())),
                                preferred_element_type=_F32)
            o_ref[:, sl] = o.astype(_BF16)

    return body


def _post_body(o_ref, x_ref, wo_ref, ln2_ref, rw_ref, x1_ref, idx_ref):
    x1 = x_ref[...] + lax.dot_general(
        o_ref[...], wo_ref[...].astype(_BF16), _NT,
        preferred_element_type=_F32)
    x1_ref[...] = x1
    var = jnp.mean(jnp.square(x1), axis=1, keepdims=True)
    h2 = (x1 * lax.rsqrt(var + EPS) * ln2_ref[...]).astype(_BF16)
    logits = lax.dot_general(h2, rw_ref[...].astype(_BF16), _NT,
                             preferred_element_type=_F32)
    mx = jnp.max(logits, axis=1, keepdims=True)
    eidx = lax.broadcasted_iota(_I32, logits.shape, 1)
    idx_ref[...] = jnp.min(jnp.where(logits == mx, eidx, E), axis=1)


def _moe_body(sb_ref, se_ref, lo_ref, hi_ref, nxe_ref, buf_ref,
              xs_ref, ln2_ref, wg_ref, wu_ref, wd_ref, out_ref,
              wbuf, wsem):
    s = pl.program_id(0)
    lo = lo_ref[s]
    hi = hi_ref[s]
    b = buf_ref[s]
    e = se_ref[s]
    nxe = nxe_ref[s]
    first = jnp.logical_or(s == 0, sb_ref[s] != sb_ref[jnp.maximum(s - 1, 0)])
    is_chg = jnp.logical_or(
        s == 0, se_ref[jnp.maximum(s - 1, 0)] != e)
    hbm = (wg_ref, wu_ref, wd_ref)

    @pl.when(s == 0)
    def _start_first():
        for m in range(3):
            pltpu.make_async_copy(hbm[m].at[e], wbuf.at[0, m],
                                  wsem.at[0, m]).start()

    @pl.when(is_chg)
    def _wait_and_prefetch():
        for m in range(3):
            pltpu.make_async_copy(hbm[m].at[e], wbuf.at[b, m],
                                  wsem.at[b, m]).wait()

    @pl.when(jnp.logical_and(is_chg, nxe != e))
    def _prefetch_next():
        for m in range(3):
            pltpu.make_async_copy(hbm[m].at[nxe], wbuf.at[1 - b, m],
                                  wsem.at[1 - b, m]).start()

    @pl.when(first)
    def _init():
        out_ref[...] = jnp.zeros_like(out_ref)

    @pl.when(hi > lo)
    def _compute():
        x = xs_ref[...]
        r = lax.broadcasted_iota(_I32, (TBLK, 1), 0)
        msk = jnp.logical_and(r >= lo, r < hi)
        xm = jnp.where(msk, x, 0.0)
        var = jnp.mean(jnp.square(xm), axis=1, keepdims=True)
        h2 = (xm * lax.rsqrt(var + EPS) * ln2_ref[...]).astype(_BF16)
        g = lax.dot_general(h2, wbuf[b, 0].astype(_BF16), _NT,
                            preferred_element_type=_F32)
        g = jnp.square(jnp.maximum(g, 0.0))
        u = lax.dot_general(h2, wbuf[b, 1].astype(_BF16), _NT,
                            preferred_element_type=_F32)
        p = (g * u).astype(_BF16)
        y = lax.dot_general(p, wbuf[b, 2].astype(_BF16), _NT,
                            preferred_element_type=_F32)
        out_ref[...] += jnp.where(msk, y + x, 0.0)


def _row_permute_kernel(scatter):
    info = plsc.get_sparse_core_info()
    nc, ns = info.num_cores, info.num_subcores
    nw = nc * ns
    rpw = S // nw
    mesh = plsc.VectorSubcoreMesh(core_axis_name="c", subcore_axis_name="s")

    @functools.partial(
        pl.kernel,
        out_type=jax.ShapeDtypeStruct((S, HIDDEN), _F32),
        mesh=mesh,
        scratch_types=[
            pltpu.VMEM((rpw,), _I32),
            pltpu.VMEM((rpw, HIDDEN), _F32),
            pltpu.SemaphoreType.DMA,
        ],
    )
    def permute(table_hbm, idx_hbm, out_hbm, idx_v, rows_v, sem):
        wid = lax.axis_index("s") * nc + lax.axis_index("c")
        base = wid * rpw
        pltpu.sync_copy(idx_hbm.at[pl.ds(base, rpw)], idx_v)
        if scatter:
            pltpu.sync_copy(table_hbm.at[pl.ds(base, rpw)], rows_v)
            pltpu.async_copy(rows_v, out_hbm.at[idx_v], sem).wait()
        else:
            pltpu.async_copy(table_hbm.at[idx_v], rows_v, sem).wait()
            pltpu.sync_copy(rows_v, out_hbm.at[pl.ds(base, rpw)])

    return permute


def _routing_metadata(idx):
    oh = (idx[:, None] == jnp.arange(E, dtype=_I32)[None, :]).astype(_I32)
    csum = jnp.cumsum(oh, axis=0)
    within = jnp.sum((csum - oh) * oh, axis=1)
    counts = csum[-1]
    offsets = jnp.concatenate(
        [jnp.zeros((1,), _I32), jnp.cumsum(counts).astype(_I32)])
    dest = jnp.take(offsets, idx) + within

    bb = jnp.arange(NB, dtype=_I32)[:, None]
    ee = jnp.arange(E, dtype=_I32)[None, :]
    st = offsets[:-1][None, :]
    en = offsets[1:][None, :]
    active = jnp.logical_and(st < (bb + 1) * TBLK, en > bb * TBLK)
    big = NB * E + 1
    key = jnp.where(active, bb * E + ee, big).reshape(-1)
    key = jnp.sort(key)[:STEPS]
    valid = key < big
    key = jnp.where(valid, key, jnp.max(jnp.where(active, bb * E + ee, -1)))
    sb = (key // E).astype(_I32)
    se = (key % E).astype(_I32)
    lo = jnp.clip(jnp.take(offsets, se) - sb * TBLK, 0, TBLK)
    hi = jnp.clip(jnp.take(offsets, se + 1) - sb * TBLK, 0, TBLK)
    lo = jnp.where(valid, lo, 0).astype(_I32)
    hi = jnp.where(valid, hi, 0).astype(_I32)

    chg = jnp.concatenate([jnp.ones((1,), _I32),
                           (se[1:] != se[:-1]).astype(_I32)])
    buf = ((jnp.cumsum(chg) - 1) % 2).astype(_I32)
    pos = jnp.where(chg == 1, jnp.arange(STEPS, dtype=_I32), STEPS)
    nxt = lax.cummin(jnp.concatenate([pos[1:], jnp.full((1,), STEPS, _I32)]),
                     axis=0, reverse=True)
    nxe = jnp.where(nxt < STEPS, se[jnp.clip(nxt, 0, STEPS - 1)], se)
    return dest, sb, se, lo, hi, nxe.astype(_I32), buf


def kernel(x, Wq, Wk, Wv, Wo, ln1, ln2, router_w, Wg, Wu, Wd):
    x2 = x.reshape(S, HIDDEN)
    ln1r = ln1.reshape(1, HIDDEN)
    ln2r = ln2.reshape(1, HIDDEN)

    row_spec = pl.BlockSpec((TROW, HIDDEN), lambda i: (i, 0))
    vec_spec = pl.BlockSpec((1, HIDDEN), lambda i: (0, 0))
    full_w = pl.BlockSpec((HIDDEN, HIDDEN), lambda i: (0, 0))

    q, k, v = pl.pallas_call(
        _qkv_body,
        grid=(S // TROW,),
        in_specs=[row_spec, vec_spec, full_w, full_w, full_w],
        out_specs=[pl.BlockSpec((TROW, HIDDEN), lambda i: (i, 0))] * 3,
        out_shape=[jax.ShapeDtypeStruct((S, HIDDEN), _BF16)] * 3,
    )(x2, ln1r, Wq, Wk, Wv)

    o_parts = []
    for pair in range(S // (2 * QBLK)):
        width = (pair + 1) * 2 * QBLK
        o_parts.append(pl.pallas_call(
            _attn_body(pair, width),
            grid=(2,),
            in_specs=[
                pl.BlockSpec((QBLK, HIDDEN),
                             lambda i, p=pair: (2 * p + i, 0)),
                pl.BlockSpec((width, HIDDEN), lambda i: (0, 0)),
                pl.BlockSpec((width, HIDDEN), lambda i: (0, 0)),
            ],
            out_specs=pl.BlockSpec((QBLK, HIDDEN), lambda i: (i, 0)),
            out_shape=jax.ShapeDtypeStruct((2 * QBLK, HIDDEN), _BF16),
        )(q, k, v))
    o = jnp.concatenate(o_parts, axis=0)

    x1, idx = pl.pallas_call(
        _post_body,
        grid=(S // TROW,),
        in_specs=[row_spec, row_spec, full_w, vec_spec,
                  pl.BlockSpec((E, HIDDEN), lambda i: (0, 0))],
        out_specs=[row_spec, pl.BlockSpec((TROW,), lambda i: (i,))],
        out_shape=[jax.ShapeDtypeStruct((S, HIDDEN), _F32),
                   jax.ShapeDtypeStruct((S,), _I32)],
    )(o, x2, Wo, ln2r, router_w)

    dest, sb, se, lo, hi, nxe, buf = _routing_metadata(idx)

    xs = _row_permute_kernel(scatter=True)(x1, dest)

    grid_spec = pltpu.PrefetchScalarGridSpec(
        num_scalar_prefetch=6,
        grid=(STEPS,),
        in_specs=[
            pl.BlockSpec((TBLK, HIDDEN),
                         lambda s, sb, se, lo, hi, nxe, buf: (sb[s], 0)),
            pl.BlockSpec((1, HIDDEN),
                         lambda s, sb, se, lo, hi, nxe, buf: (0, 0)),
            pl.BlockSpec(memory_space=pl.ANY),
            pl.BlockSpec(memory_space=pl.ANY),
            pl.BlockSpec(memory_space=pl.ANY),
        ],
        out_specs=pl.BlockSpec((TBLK, HIDDEN),
                               lambda s, sb, se, lo, hi, nxe, buf: (sb[s], 0)),
        scratch_shapes=[
            pltpu.VMEM((2, 3, HIDDEN, HIDDEN), _F32),
            pltpu.SemaphoreType.DMA((2, 3)),
        ],
    )
    ys = pl.pallas_call(
        _moe_body,
        grid_spec=grid_spec,
        out_shape=jax.ShapeDtypeStruct((S, HIDDEN), _F32),
    )(sb, se, lo, hi, nxe, buf, xs, ln2r, Wg, Wu, Wd)

    out = _row_permute_kernel(scatter=False)(ys, dest)
    return out.reshape(B, S, HIDDEN)
